# baseline (speedup 1.0000x reference)
import functools

import jax
import jax.numpy as jnp
from jax import lax
from jax.experimental import pallas as pl
from jax.experimental.pallas import tpu as pltpu
from jax.experimental.pallas import tpu_sc as plsc

B = 16384
D = 32
NUM_NS = 4
NCTX = NUM_NS + 1

NC = 2
NS = 16
NW = NC * NS
BPW = B // NW
CPW = BPW * NCTX
G = 128
TG = BPW // G
CG = CPW // G
OPAD = 8

_GDN = lax.GatherDimensionNumbers(
    offset_dims=(), collapsed_slice_dims=(0,), start_index_map=(0,))


def _permute(v, idx):
  return lax.gather(v, idx.reshape(16, 1), _GDN, (1,),
                    mode=lax.GatherScatterMode.PROMISE_IN_BOUNDS)


def _body(tt_hbm, tidx_hbm, ct_hbm, cidx_hbm, out_hbm,
          tidx_v, cidx_v, rows_t, rows_c, out_v, sem):
  cid = lax.axis_index("c")
  sid = lax.axis_index("s")
  wid = cid * NS + sid

  pltpu.sync_copy(tidx_hbm.at[pl.ds(wid * BPW, BPW)], tidx_v)
  pltpu.sync_copy(cidx_hbm.at[pl.ds(wid * CPW, CPW)], cidx_v)

  copies = []
  for g in range(TG):
    copies.append(
        pltpu.async_copy(tt_hbm.at[tidx_v.at[pl.ds(g * G, G)]],
                         rows_t.at[pl.ds(g * G, G)], sem))
  for g in range(CG):
    copies.append(
        pltpu.async_copy(ct_hbm.at[cidx_v.at[pl.ds(g * G, G)]],
                         rows_c.at[pl.ds(g * G, G)], sem))
  for cp in copies:
    cp.wait()

  iota16 = lax.broadcasted_iota(jnp.int32, (16,), 0)
  perms = [iota16 ^ sh for sh in (8, 4, 2, 1)]
  lane_masks = [iota16 == k for k in range(16)]

  def hsum(v):
    for p in perms:
      v = v + _permute(v, p)
    return v

  def body(i, carry):
    acc = jnp.zeros((16,), jnp.float32)
    for half in range(2):
      b = 2 * i + half
      w0 = rows_t[b, pl.ds(0, 16)]
      w1 = rows_t[b, pl.ds(16, 16)]
      for n in range(NCTX):
        r = b * NCTX + n
        c0 = rows_c[r, pl.ds(0, 16)]
        c1 = rows_c[r, pl.ds(16, 16)]
        s = hsum(c0 * w0 + c1 * w1)
        acc = jnp.where(lane_masks[half * 8 + n], s, acc)
    out_v[pl.ds(i * 16, 16)] = acc
    return carry

  lax.fori_loop(0, BPW // 2, body, 0)

  pltpu.sync_copy(out_v, out_hbm.at[pl.ds(wid * BPW * OPAD, BPW * OPAD)])


@jax.jit
def kernel(target, context, target_table, context_table):
  tidx = target.reshape(B)
  cidx = context.reshape(B * NCTX)

  mesh = plsc.VectorSubcoreMesh(core_axis_name="c", subcore_axis_name="s")
  run = pl.kernel(
      _body,
      out_type=jax.ShapeDtypeStruct((B * OPAD,), jnp.float32),
      mesh=mesh,
      scratch_types=[
          pltpu.VMEM((BPW,), jnp.int32),
          pltpu.VMEM((CPW,), jnp.int32),
          pltpu.VMEM((BPW, D), jnp.float32),
          pltpu.VMEM((CPW, D), jnp.float32),
          pltpu.VMEM((BPW * OPAD,), jnp.float32),
          pltpu.SemaphoreType.DMA,
      ],
      compiler_params=pltpu.CompilerParams(use_tc_tiling_on_sc=False),
  )
  out_flat = run(target_table, tidx, context_table, cidx)
  return out_flat.reshape(B, OPAD)[:, :NCTX]
